# baseline (speedup 1.0000x reference)
import functools

import jax
import jax.numpy as jnp
from jax import lax
from jax.experimental import pallas as pl
from jax.experimental.pallas import tpu as pltpu
from jax.experimental.pallas import tpu_sc as plsc

N_CLS = 1024
L_SUF = 61
L_PAD = 64
N_CTX = 16
CTX_LEN = 77
D = 512
NC, NS = 2, 16
NW = NC * NS
CPW = N_CLS // NW


def _body(prompt_hbm, promptT_hbm, ctx_hbm, table_hbm, out_hbm, eos_hbm,
          pbuf, pbufT, rows_v, ctx_v, eos_v, gsem):
    wid = lax.axis_index("s") * NC + lax.axis_index("c")
    base = wid * CPW
    pltpu.sync_copy(prompt_hbm.at[pl.ds(base, CPW)], pbuf)
    pltpu.sync_copy(promptT_hbm.at[:, pl.ds(base, CPW)], pbufT)

    for h in range(CPW // 16):
        def amax(j, carry):
            best, besti = carry
            cur = pbufT[j, pl.ds(h * 16, 16)]
            m = cur > best
            return jnp.where(m, cur, best), jnp.where(m, jnp.full((16,), 1, jnp.int32) * j, besti)

        init = (jnp.full((16,), -1, jnp.int32), jnp.zeros((16,), jnp.int32))
        _, besti = lax.fori_loop(0, L_SUF, amax, init)
        eos_v[pl.ds(h * 16, 16)] = besti + N_CTX
    pltpu.sync_copy(eos_v, eos_hbm.at[pl.ds(base, CPW)])

    def cls_body(i, _):
        c = base + i
        idx = pbuf.at[i]
        pltpu.async_copy(table_hbm.at[idx], rows_v, gsem).wait()
        pltpu.sync_copy(ctx_hbm.at[pl.ds(c * N_CTX, N_CTX)], ctx_v)
        o = c * CTX_LEN
        pltpu.sync_copy(rows_v.at[pl.ds(0, 1)], out_hbm.at[pl.ds(o, 1)])
        pltpu.sync_copy(ctx_v, out_hbm.at[pl.ds(o + 1, N_CTX)])
        pltpu.sync_copy(rows_v.at[pl.ds(1, L_SUF - 1)],
                        out_hbm.at[pl.ds(o + 1 + N_CTX, L_SUF - 1)])
        return 0

    lax.fori_loop(0, CPW, cls_body, 0)


@functools.partial(
    pl.kernel,
    mesh=plsc.VectorSubcoreMesh(core_axis_name="c", subcore_axis_name="s"),
    compiler_params=pltpu.CompilerParams(use_tc_tiling_on_sc=False),
    out_type=[
        jax.ShapeDtypeStruct((N_CLS * CTX_LEN, D), jnp.float32),
        jax.ShapeDtypeStruct((N_CLS,), jnp.int32),
    ],
    scratch_types=[
        pltpu.VMEM((CPW, L_PAD), jnp.int32),
        pltpu.VMEM((L_SUF, CPW), jnp.int32),
        pltpu.VMEM((L_PAD, D), jnp.float32),
        pltpu.VMEM((N_CTX, D), jnp.float32),
        pltpu.VMEM((CPW,), jnp.int32),
        pltpu.SemaphoreType.DMA,
    ],
)
def _prompt_kernel(prompt_hbm, promptT_hbm, ctx_hbm, table_hbm, out_hbm, eos_hbm,
                   pbuf, pbufT, rows_v, ctx_v, eos_v, gsem):
    _body(prompt_hbm, promptT_hbm, ctx_hbm, table_hbm, out_hbm, eos_hbm,
          pbuf, pbufT, rows_v, ctx_v, eos_v, gsem)


def kernel(prompt, ctx_embedding, token_embedding):
    prompt_pad = jnp.pad(prompt, ((0, 0), (0, L_PAD - L_SUF)))
    ctx2 = ctx_embedding.reshape(N_CLS * N_CTX, D)
    out2, eos = _prompt_kernel(prompt_pad, prompt.T, ctx2, token_embedding)
    return out2.reshape(N_CLS, CTX_LEN, D), eos
